# baseline (device time: 78105 ns/iter reference)
import jax
import jax.numpy as jnp
from jax import lax
from jax.experimental import pallas as pl
from jax.experimental.pallas import tpu as pltpu

N_DEV = 32
M_BLK = 128
K = 4096
N = 8192
K_TILE = 128
N_STEPS = K // K_TILE

FP8 = jnp.float8_e5m2
BF16 = jnp.bfloat16


def kernel(x, w_mat, scale_x, scale_w):
    def body(x_ref, w_hbm, sx_ref, sw_ref, out_ref,
             xs_ref, xt_ref, w_vmem, send_sems, recv_sems, wsems):
        t = pl.program_id(0)
        my = lax.axis_index("i")

        def src_of(step):
            return lax.rem(my - 1 - step + 2 * N_DEV, N_DEV)

        def wtile_copy(step, slot):
            j = src_of(step)
            return pltpu.make_async_copy(
                w_hbm.at[pl.ds(j * K_TILE, K_TILE), :],
                w_vmem.at[slot],
                wsems.at[slot],
            )

        @pl.when(t == 0)
        def _():
            xs_ref[...] = x_ref[...].astype(FP8)
            xt_ref[:, pl.ds(my * M_BLK, M_BLK)] = xs_ref[pl.ds(my * M_BLK, M_BLK), :]
            for off in range(1, N_DEV):
                d = lax.rem(my + off, N_DEV)
                pltpu.make_async_remote_copy(
                    src_ref=xs_ref.at[pl.ds(d * M_BLK, M_BLK), :],
                    dst_ref=xt_ref.at[:, pl.ds(my * M_BLK, M_BLK)],
                    send_sem=send_sems.at[off],
                    recv_sem=recv_sems.at[my],
                    device_id=(d,),
                    device_id_type=pl.DeviceIdType.MESH,
                ).start()
            wtile_copy(0, 0).start()
            wtile_copy(1, 1).start()

        j = src_of(t)
        slot = lax.rem(t, 2)

        @pl.when(t < N_STEPS - 1)
        def _():
            pltpu.make_async_remote_copy(
                src_ref=xs_ref.at[pl.ds(j * M_BLK, M_BLK), :],
                dst_ref=xt_ref.at[:, pl.ds(j * M_BLK, M_BLK)],
                send_sem=send_sems.at[0],
                recv_sem=recv_sems.at[j],
                device_id=(j,),
                device_id_type=pl.DeviceIdType.MESH,
            ).wait_recv()

        wtile_copy(t, slot).wait()

        xpart = xt_ref[:, pl.ds(j * M_BLK, M_BLK)].astype(BF16)
        wq = w_vmem[slot].astype(BF16)
        acc = lax.dot_general(
            xpart, wq,
            dimension_numbers=(((1,), (0,)), ((), ())),
            preferred_element_type=jnp.float32,
        )

        @pl.when(t == 0)
        def _():
            out_ref[...] = acc

        @pl.when(t > 0)
        def _():
            out_ref[...] += acc

        @pl.when(t + 2 < N_STEPS)
        def _():
            wtile_copy(t + 2, slot).start()

        @pl.when(t == N_STEPS - 1)
        def _():
            y = out_ref[...] * (sx_ref[0] * sw_ref[0])
            out_ref[...] = y * (1.0 / (1.0 + jnp.exp(-jnp.clip(y, -60.0, 60.0))))
            for off in range(1, N_DEV):
                d = lax.rem(my + off, N_DEV)
                pltpu.make_async_remote_copy(
                    src_ref=xs_ref.at[pl.ds(d * M_BLK, M_BLK), :],
                    dst_ref=xt_ref.at[:, pl.ds(my * M_BLK, M_BLK)],
                    send_sem=send_sems.at[off],
                    recv_sem=recv_sems.at[my],
                    device_id=(d,),
                    device_id_type=pl.DeviceIdType.MESH,
                ).wait_send()

    return pl.pallas_call(
        body,
        grid=(N_STEPS,),
        in_specs=[
            pl.BlockSpec((K, M_BLK), lambda t: (0, 0)),
            pl.BlockSpec(memory_space=pltpu.MemorySpace.HBM),
            pl.BlockSpec(memory_space=pltpu.SMEM),
            pl.BlockSpec(memory_space=pltpu.SMEM),
        ],
        out_specs=pl.BlockSpec((M_BLK, N), lambda t: (0, 0)),
        out_shape=jax.ShapeDtypeStruct((M_BLK, N), jnp.float32),
        scratch_shapes=[
            pltpu.VMEM((K, M_BLK), FP8),
            pltpu.VMEM((M_BLK, K), FP8),
            pltpu.VMEM((2, K_TILE, N), jnp.float32),
            pltpu.SemaphoreType.DMA((N_DEV,)),
            pltpu.SemaphoreType.DMA((N_DEV,)),
            pltpu.SemaphoreType.DMA((2,)),
        ],
        compiler_params=pltpu.CompilerParams(
            dimension_semantics=("arbitrary",),
        ),
    )(x, w_mat, scale_x, scale_w)


# device time: 68421 ns/iter; 1.1415x vs baseline; 1.1415x over previous
import jax
import jax.numpy as jnp
from jax import lax
from jax.experimental import pallas as pl
from jax.experimental.pallas import tpu as pltpu

N_DEV = 32
M_BLK = 128
K = 4096
N = 8192
N_TILE = 1024
N_TILES = N // N_TILE

FP8 = jnp.float8_e5m2
BF16 = jnp.bfloat16


def kernel(x, w_mat, scale_x, scale_w):
    def body(x_ref, w_ref, sx_ref, sw_ref, out_ref,
             xs_ref, xt_ref, send_sems, recv_sems):
        t = pl.program_id(0)
        my = lax.axis_index("i")

        @pl.when(t == 0)
        def _():
            xs_ref[...] = x_ref[...].astype(FP8)
            xt_ref[:, pl.ds(my * M_BLK, M_BLK)] = xs_ref[pl.ds(my * M_BLK, M_BLK), :]

            for off in range(1, N_DEV):
                d = lax.rem(my + off, N_DEV)
                pltpu.make_async_remote_copy(
                    src_ref=xs_ref.at[pl.ds(d * M_BLK, M_BLK), :],
                    dst_ref=xt_ref.at[:, pl.ds(my * M_BLK, M_BLK)],
                    send_sem=send_sems.at[off],
                    recv_sem=recv_sems.at[my],
                    device_id=(d,),
                    device_id_type=pl.DeviceIdType.MESH,
                ).start()

            for off in range(1, N_DEV):
                j = lax.rem(my + off, N_DEV)
                pltpu.make_async_remote_copy(
                    src_ref=xs_ref.at[pl.ds(j * M_BLK, M_BLK), :],
                    dst_ref=xt_ref.at[:, pl.ds(j * M_BLK, M_BLK)],
                    send_sem=send_sems.at[off],
                    recv_sem=recv_sems.at[j],
                    device_id=(j,),
                    device_id_type=pl.DeviceIdType.MESH,
                ).wait_recv()

            for off in range(1, N_DEV):
                d = lax.rem(my + off, N_DEV)
                pltpu.make_async_remote_copy(
                    src_ref=xs_ref.at[pl.ds(d * M_BLK, M_BLK), :],
                    dst_ref=xt_ref.at[:, pl.ds(my * M_BLK, M_BLK)],
                    send_sem=send_sems.at[off],
                    recv_sem=recv_sems.at[my],
                    device_id=(d,),
                    device_id_type=pl.DeviceIdType.MESH,
                ).wait_send()

        wq = w_ref[...].astype(BF16)
        acc = lax.dot_general(
            xt_ref[...].astype(BF16),
            wq,
            dimension_numbers=(((1,), (0,)), ((), ())),
            preferred_element_type=jnp.float32,
        )
        y = acc * (sx_ref[0] * sw_ref[0])
        out_ref[...] = y * (1.0 / (1.0 + jnp.exp(-jnp.clip(y, -60.0, 60.0))))

    return pl.pallas_call(
        body,
        grid=(N_TILES,),
        in_specs=[
            pl.BlockSpec((K, M_BLK), lambda t: (0, 0)),
            pl.BlockSpec((K, N_TILE), lambda t: (0, t)),
            pl.BlockSpec(memory_space=pltpu.SMEM),
            pl.BlockSpec(memory_space=pltpu.SMEM),
        ],
        out_specs=pl.BlockSpec((M_BLK, N_TILE), lambda t: (0, t)),
        out_shape=jax.ShapeDtypeStruct((M_BLK, N), jnp.float32),
        scratch_shapes=[
            pltpu.VMEM((K, M_BLK), FP8),
            pltpu.VMEM((M_BLK, K), FP8),
            pltpu.SemaphoreType.DMA((N_DEV,)),
            pltpu.SemaphoreType.DMA((N_DEV,)),
        ],
        compiler_params=pltpu.CompilerParams(
            dimension_semantics=("arbitrary",),
            vmem_limit_bytes=60 * 1024 * 1024,
        ),
    )(x, w_mat, scale_x, scale_w)


# device time: 64464 ns/iter; 1.2116x vs baseline; 1.0614x over previous
import jax
import jax.numpy as jnp
from jax import lax
from jax.experimental import pallas as pl
from jax.experimental.pallas import tpu as pltpu

N_DEV = 32
M_BLK = 128
K = 4096
N = 8192
N_TILE = 512
N_TILES = N // N_TILE

FP8 = jnp.float8_e5m2


def kernel(x, w_mat, scale_x, scale_w):
    def body(x_ref, w_ref, sx_ref, sw_ref, out_ref,
             xs_ref, xt_ref, send_sems, recv_sems):
        t = pl.program_id(0)
        my = lax.axis_index("i")

        @pl.when(t == 0)
        def _():
            xs_ref[...] = x_ref[...].astype(FP8)
            xt_ref[:, pl.ds(my * M_BLK, M_BLK)] = xs_ref[pl.ds(my * M_BLK, M_BLK), :]

            for off in range(1, N_DEV):
                d = lax.rem(my + off, N_DEV)
                send = pltpu.make_async_remote_copy(
                    src_ref=xs_ref.at[pl.ds(d * M_BLK, M_BLK), :],
                    dst_ref=xt_ref.at[:, pl.ds(my * M_BLK, M_BLK)],
                    send_sem=send_sems.at[off],
                    recv_sem=recv_sems.at[my],
                    device_id=(d,),
                    device_id_type=pl.DeviceIdType.MESH,
                )
                send.start()

            for off in range(1, N_DEV):
                j = lax.rem(my + off, N_DEV)
                recv = pltpu.make_async_remote_copy(
                    src_ref=xs_ref.at[pl.ds(j * M_BLK, M_BLK), :],
                    dst_ref=xt_ref.at[:, pl.ds(j * M_BLK, M_BLK)],
                    send_sem=send_sems.at[off],
                    recv_sem=recv_sems.at[j],
                    device_id=(j,),
                    device_id_type=pl.DeviceIdType.MESH,
                )
                recv.wait_recv()

            for off in range(1, N_DEV):
                d = lax.rem(my + off, N_DEV)
                send = pltpu.make_async_remote_copy(
                    src_ref=xs_ref.at[pl.ds(d * M_BLK, M_BLK), :],
                    dst_ref=xt_ref.at[:, pl.ds(my * M_BLK, M_BLK)],
                    send_sem=send_sems.at[off],
                    recv_sem=recv_sems.at[my],
                    device_id=(d,),
                    device_id_type=pl.DeviceIdType.MESH,
                )
                send.wait_send()

        wq = w_ref[...].astype(jnp.bfloat16)
        acc = lax.dot_general(
            xt_ref[...].astype(jnp.bfloat16),
            wq,
            dimension_numbers=(((1,), (0,)), ((), ())),
            preferred_element_type=jnp.float32,
        )
        y = acc * (sx_ref[0] * sw_ref[0])
        out_ref[...] = y * (1.0 / (1.0 + jnp.exp(-jnp.clip(y, -60.0, 60.0))))

    return pl.pallas_call(
        body,
        grid=(N_TILES,),
        in_specs=[
            pl.BlockSpec((K, M_BLK), lambda t: (0, 0)),
            pl.BlockSpec((K, N_TILE), lambda t: (0, t)),
            pl.BlockSpec(memory_space=pltpu.SMEM),
            pl.BlockSpec(memory_space=pltpu.SMEM),
        ],
        out_specs=pl.BlockSpec((M_BLK, N_TILE), lambda t: (0, t)),
        out_shape=jax.ShapeDtypeStruct((M_BLK, N), jnp.float32),
        scratch_shapes=[
            pltpu.VMEM((K, M_BLK), FP8),
            pltpu.VMEM((M_BLK, K), FP8),
            pltpu.SemaphoreType.DMA((N_DEV,)),
            pltpu.SemaphoreType.DMA((N_DEV,)),
        ],
        compiler_params=pltpu.CompilerParams(
            dimension_semantics=("arbitrary",),
        ),
    )(x, w_mat, scale_x, scale_w)
